# baseline (speedup 1.0000x reference)
import functools

import jax
import jax.numpy as jnp
from jax import lax
from jax.experimental import pallas as pl
from jax.experimental.pallas import tpu as pltpu
from jax.experimental.pallas import tpu_sc as plsc

_NC = 2
_NS = 16
_NW = _NC * _NS
_CHUNK = 128


def _make_sc_gather(batch, emb):
    b_per_w = batch // _NW
    n_chunks = b_per_w // _CHUNK
    mesh = plsc.VectorSubcoreMesh(core_axis_name="c", subcore_axis_name="s")

    @functools.partial(
        pl.kernel,
        mesh=mesh,
        out_type=(
            jax.ShapeDtypeStruct((batch, emb), jnp.float32),
            jax.ShapeDtypeStruct((batch, emb), jnp.float32),
        ),
        scratch_types=[
            pltpu.VMEM((n_chunks, _CHUNK), jnp.int32),
            pltpu.VMEM((n_chunks, _CHUNK), jnp.int32),
            pltpu.VMEM((b_per_w, emb), jnp.float32),
            pltpu.VMEM((b_per_w, emb), jnp.float32),
            pltpu.SemaphoreType.DMA,
            pltpu.SemaphoreType.DMA,
        ],
        compiler_params=pltpu.CompilerParams(use_tc_tiling_on_sc=False),
    )
    def gather_kernel(u_idx_hbm, i_idx_hbm, u_tab_hbm, i_tab_hbm,
                      u_out_hbm, i_out_hbm,
                      uidx_v, iidx_v, urows_v, irows_v, usem, isem):
        wid = lax.axis_index("s") * _NC + lax.axis_index("c")
        base = wid * b_per_w
        pltpu.sync_copy(u_idx_hbm.at[wid], uidx_v)
        pltpu.sync_copy(i_idx_hbm.at[wid], iidx_v)
        ucopies = [
            pltpu.async_copy(u_tab_hbm.at[uidx_v.at[j]],
                             urows_v.at[pl.ds(j * _CHUNK, _CHUNK)], usem)
            for j in range(n_chunks)
        ]
        icopies = [
            pltpu.async_copy(i_tab_hbm.at[iidx_v.at[j]],
                             irows_v.at[pl.ds(j * _CHUNK, _CHUNK)], isem)
            for j in range(n_chunks)
        ]
        for c in ucopies:
            c.wait()
        pltpu.sync_copy(urows_v, u_out_hbm.at[pl.ds(base, b_per_w)])
        for c in icopies:
            c.wait()
        pltpu.sync_copy(irows_v, i_out_hbm.at[pl.ds(base, b_per_w)])

    return gather_kernel


def _mlp_body(u_ref, i_ref, w1u_ref, w1i_ref, b1_ref, w2_ref, b2_ref,
              w3_ref, b3_ref, out_ref):
    h = jnp.dot(u_ref[...], w1u_ref[...],
                preferred_element_type=jnp.float32,
                precision=lax.Precision.HIGHEST)
    h = h + jnp.dot(i_ref[...], w1i_ref[...],
                    preferred_element_type=jnp.float32,
                    precision=lax.Precision.HIGHEST)
    h = jnp.maximum(h + b1_ref[...], 0.0)
    h = jnp.maximum(
        jnp.dot(h, w2_ref[...], preferred_element_type=jnp.float32,
                precision=lax.Precision.HIGHEST) + b2_ref[...], 0.0)
    out_ref[...] = jnp.dot(h, w3_ref[...], preferred_element_type=jnp.float32,
                           precision=lax.Precision.HIGHEST) + b3_ref[...]


def kernel(userIDs, itemIDs, user_table, item_table, W1, b1, W2, b2, W3, b3):
    batch = userIDs.shape[0]
    emb = user_table.shape[1]
    b_per_w = batch // _NW
    n_chunks = b_per_w // _CHUNK

    u_idx3 = userIDs.astype(jnp.int32).reshape(_NW, n_chunks, _CHUNK)
    i_idx3 = itemIDs.astype(jnp.int32).reshape(_NW, n_chunks, _CHUNK)

    u_rows, i_rows = _make_sc_gather(batch, emb)(
        u_idx3, i_idx3, user_table, item_table)

    blk = 2048
    grid = batch // blk
    hid1 = W1.shape[1]
    hid2 = W2.shape[1]
    w1u = W1[:emb]
    w1i = W1[emb:]

    out = pl.pallas_call(
        _mlp_body,
        grid=(grid,),
        in_specs=[
            pl.BlockSpec((blk, emb), lambda i: (i, 0)),
            pl.BlockSpec((blk, emb), lambda i: (i, 0)),
            pl.BlockSpec((emb, hid1), lambda i: (0, 0)),
            pl.BlockSpec((emb, hid1), lambda i: (0, 0)),
            pl.BlockSpec((1, hid1), lambda i: (0, 0)),
            pl.BlockSpec((hid1, hid2), lambda i: (0, 0)),
            pl.BlockSpec((1, hid2), lambda i: (0, 0)),
            pl.BlockSpec((hid2, 1), lambda i: (0, 0)),
            pl.BlockSpec((1, 1), lambda i: (0, 0)),
        ],
        out_specs=pl.BlockSpec((blk, 1), lambda i: (i, 0)),
        out_shape=jax.ShapeDtypeStruct((batch, 1), jnp.float32),
    )(u_rows, i_rows, w1u, w1i, b1.reshape(1, hid1), W2,
      b2.reshape(1, hid2), W3, b3.reshape(1, 1))
    return out


# speedup vs baseline: 1.6871x; 1.6871x over previous
import functools

import jax
import jax.numpy as jnp
from jax import lax
from jax.experimental import pallas as pl
from jax.experimental.pallas import tpu as pltpu
from jax.experimental.pallas import tpu_sc as plsc

_NC = 2
_NS = 16
_NW = _NC * _NS
_ICHUNK = 128


def _transpose_body(u_ref, i_ref, out_ref):
    out_ref[:, : u_ref.shape[0]] = u_ref[...].T
    out_ref[:, u_ref.shape[0]:] = i_ref[...].T


def _make_sc_gather(batch, width, vocab):
    b_per_w = batch // _NW
    n_chunks = b_per_w // _ICHUNK
    mesh = plsc.VectorSubcoreMesh(core_axis_name="c", subcore_axis_name="s")

    @functools.partial(
        pl.kernel,
        mesh=mesh,
        out_type=jax.ShapeDtypeStruct((batch, width), jnp.float32),
        scratch_types=[
            pltpu.VMEM((n_chunks, _ICHUNK), jnp.int32),
            pltpu.VMEM((b_per_w, width), jnp.float32),
            pltpu.SemaphoreType.DMA,
        ],
    )
    def gather_kernel(idx_hbm, tab_hbm, out_hbm, idx_v, rows_v, sem):
        wid = lax.axis_index("s") * _NC + lax.axis_index("c")
        base = wid * b_per_w
        pltpu.sync_copy(idx_hbm.at[wid], idx_v)
        copies = [
            pltpu.make_async_copy(
                tab_hbm.at[idx_v.at[k]],
                rows_v.at[pl.ds(k * _ICHUNK, _ICHUNK)], sem)
            for k in range(n_chunks)
        ]
        for c in copies:
            c.start()
        for c in copies:
            c.wait()
        pltpu.sync_copy(rows_v, out_hbm.at[pl.ds(base, b_per_w)])

    return gather_kernel


def kernel(userIDs, itemIDs, user_table, item_table, W1, b1, W2, b2, W3, b3):
    batch = userIDs.shape[0]
    emb = user_table.shape[1]
    vocab = user_table.shape[0]
    width = 2 * emb
    b_per_w = batch // _NW

    cblk = 2048
    tgrid = (vocab + cblk - 1) // cblk
    cat_rm = pl.pallas_call(
        _transpose_body,
        grid=(tgrid,),
        in_specs=[
            pl.BlockSpec((emb, cblk), lambda i: (0, i)),
            pl.BlockSpec((emb, cblk), lambda i: (0, i)),
        ],
        out_specs=pl.BlockSpec((cblk, width), lambda i: (i, 0)),
        out_shape=jax.ShapeDtypeStruct((vocab, width), jnp.float32),
    )(user_table.T, item_table.T)

    u_idx3 = userIDs.astype(jnp.int32).reshape(_NW, b_per_w // _ICHUNK,
                                               _ICHUNK)
    i_idx3 = itemIDs.astype(jnp.int32).reshape(_NW, b_per_w // _ICHUNK,
                                               _ICHUNK)
    gather = _make_sc_gather(batch, width, vocab)
    u_rows = gather(u_idx3, cat_rm)
    i_rows = gather(i_idx3, cat_rm)

    hid1 = W1.shape[1]
    hid2 = W2.shape[1]

    blk = 2048
    grid = batch // blk
    w1u = jnp.concatenate([W1[:emb], jnp.zeros_like(W1[emb:])], axis=0)
    w1i = jnp.concatenate([jnp.zeros_like(W1[:emb]), W1[emb:]], axis=0)

    def _mlp2_body(u_ref, i_ref, w1u_ref, w1i_ref, b1_ref, w2_ref, b2_ref,
                   w3_ref, b3_ref, out_ref):
        h = jnp.dot(u_ref[...], w1u_ref[...],
                    preferred_element_type=jnp.float32)
        h = h + jnp.dot(i_ref[...], w1i_ref[...],
                        preferred_element_type=jnp.float32)
        h = jnp.maximum(h + b1_ref[...], 0.0)
        h = jnp.maximum(
            jnp.dot(h, w2_ref[...], preferred_element_type=jnp.float32) + b2_ref[...], 0.0)
        out_ref[...] = jnp.dot(
            h, w3_ref[...], preferred_element_type=jnp.float32) + b3_ref[...]

    out = pl.pallas_call(
        _mlp2_body,
        grid=(grid,),
        in_specs=[
            pl.BlockSpec((blk, width), lambda i: (i, 0)),
            pl.BlockSpec((blk, width), lambda i: (i, 0)),
            pl.BlockSpec((width, hid1), lambda i: (0, 0)),
            pl.BlockSpec((width, hid1), lambda i: (0, 0)),
            pl.BlockSpec((1, hid1), lambda i: (0, 0)),
            pl.BlockSpec((hid1, hid2), lambda i: (0, 0)),
            pl.BlockSpec((1, hid2), lambda i: (0, 0)),
            pl.BlockSpec((hid2, 1), lambda i: (0, 0)),
            pl.BlockSpec((1, 1), lambda i: (0, 0)),
        ],
        out_specs=pl.BlockSpec((blk, 1), lambda i: (i, 0)),
        out_shape=jax.ShapeDtypeStruct((batch, 1), jnp.float32),
    )(u_rows, i_rows, w1u, w1i, b1.reshape(1, hid1), W2,
      b2.reshape(1, hid2), W3, b3.reshape(1, 1))
    return out


# speedup vs baseline: 2.0994x; 1.2444x over previous
import functools

import jax
import jax.numpy as jnp
from jax import lax
from jax.experimental import pallas as pl
from jax.experimental.pallas import tpu as pltpu
from jax.experimental.pallas import tpu_sc as plsc

_NC = 2
_NS = 16
_NW = _NC * _NS
_ICHUNK = 128


def _transpose_body(u_ref, i_ref, out_ref):
    eye = jnp.eye(u_ref.shape[0], dtype=jnp.float32)
    dn = (((0,), (0,)), ((), ()))
    out_ref[:, : u_ref.shape[0]] = lax.dot_general(
        u_ref[...], eye, dn, preferred_element_type=jnp.float32)
    out_ref[:, u_ref.shape[0]:] = lax.dot_general(
        i_ref[...], eye, dn, preferred_element_type=jnp.float32)


def _make_sc_gather(batch, width, vocab):
    b_per_w = batch // _NW
    n_chunks = b_per_w // _ICHUNK
    mesh = plsc.VectorSubcoreMesh(core_axis_name="c", subcore_axis_name="s")

    @functools.partial(
        pl.kernel,
        mesh=mesh,
        out_type=jax.ShapeDtypeStruct((batch, width), jnp.float32),
        scratch_types=[
            pltpu.VMEM((n_chunks, _ICHUNK), jnp.int32),
            pltpu.VMEM((b_per_w, width), jnp.float32),
            pltpu.SemaphoreType.DMA,
        ],
    )
    def gather_kernel(idx_hbm, tab_hbm, out_hbm, idx_v, rows_v, sem):
        wid = lax.axis_index("s") * _NC + lax.axis_index("c")
        base = wid * b_per_w
        pltpu.sync_copy(idx_hbm.at[wid], idx_v)
        copies = [
            pltpu.make_async_copy(
                tab_hbm.at[idx_v.at[k]],
                rows_v.at[pl.ds(k * _ICHUNK, _ICHUNK)], sem)
            for k in range(n_chunks)
        ]
        for c in copies:
            c.start()
        for c in copies:
            c.wait()
        pltpu.sync_copy(rows_v, out_hbm.at[pl.ds(base, b_per_w)])

    return gather_kernel


def kernel(userIDs, itemIDs, user_table, item_table, W1, b1, W2, b2, W3, b3):
    batch = userIDs.shape[0]
    emb = user_table.shape[1]
    vocab = user_table.shape[0]
    width = 2 * emb
    b_per_w = batch // _NW

    cblk = 4096
    tgrid = (vocab + cblk - 1) // cblk
    cat_rm = pl.pallas_call(
        _transpose_body,
        grid=(tgrid,),
        in_specs=[
            pl.BlockSpec((emb, cblk), lambda i: (0, i)),
            pl.BlockSpec((emb, cblk), lambda i: (0, i)),
        ],
        out_specs=pl.BlockSpec((cblk, width), lambda i: (i, 0)),
        out_shape=jax.ShapeDtypeStruct((vocab, width), jnp.float32),
    )(user_table.T, item_table.T)

    u_idx3 = userIDs.astype(jnp.int32).reshape(_NW, b_per_w // _ICHUNK,
                                               _ICHUNK)
    i_idx3 = itemIDs.astype(jnp.int32).reshape(_NW, b_per_w // _ICHUNK,
                                               _ICHUNK)
    gather = _make_sc_gather(batch, width, vocab)
    u_rows = gather(u_idx3, cat_rm)
    i_rows = gather(i_idx3, cat_rm)

    hid1 = W1.shape[1]
    hid2 = W2.shape[1]

    blk = 2048
    grid = batch // blk
    w1u = jnp.concatenate([W1[:emb], jnp.zeros_like(W1[emb:])], axis=0)
    w1i = jnp.concatenate([jnp.zeros_like(W1[:emb]), W1[emb:]], axis=0)

    def _mlp2_body(u_ref, i_ref, w1u_ref, w1i_ref, b1_ref, w2_ref, b2_ref,
                   w3_ref, b3_ref, out_ref):
        h = jnp.dot(u_ref[...], w1u_ref[...],
                    preferred_element_type=jnp.float32)
        h = h + jnp.dot(i_ref[...], w1i_ref[...],
                        preferred_element_type=jnp.float32)
        h = jnp.maximum(h + b1_ref[...], 0.0)
        h = jnp.maximum(
            jnp.dot(h, w2_ref[...], preferred_element_type=jnp.float32) + b2_ref[...], 0.0)
        out_ref[...] = jnp.dot(
            h, w3_ref[...], preferred_element_type=jnp.float32) + b3_ref[...]

    out = pl.pallas_call(
        _mlp2_body,
        grid=(grid,),
        in_specs=[
            pl.BlockSpec((blk, width), lambda i: (i, 0)),
            pl.BlockSpec((blk, width), lambda i: (i, 0)),
            pl.BlockSpec((width, hid1), lambda i: (0, 0)),
            pl.BlockSpec((width, hid1), lambda i: (0, 0)),
            pl.BlockSpec((1, hid1), lambda i: (0, 0)),
            pl.BlockSpec((hid1, hid2), lambda i: (0, 0)),
            pl.BlockSpec((1, hid2), lambda i: (0, 0)),
            pl.BlockSpec((hid2, 1), lambda i: (0, 0)),
            pl.BlockSpec((1, 1), lambda i: (0, 0)),
        ],
        out_specs=pl.BlockSpec((blk, 1), lambda i: (i, 0)),
        out_shape=jax.ShapeDtypeStruct((batch, 1), jnp.float32),
    )(u_rows, i_rows, w1u, w1i, b1.reshape(1, hid1), W2,
      b2.reshape(1, hid2), W3, b3.reshape(1, 1))
    return out


# speedup vs baseline: 3.1060x; 1.4794x over previous
import functools

import jax
import jax.numpy as jnp
from jax import lax
from jax.experimental import pallas as pl
from jax.experimental.pallas import tpu as pltpu
from jax.experimental.pallas import tpu_sc as plsc

_NC = 2
_NS = 16
_NW = _NC * _NS
_ICHUNK = 128


def _transpose_body(u_ref, i_ref, out_ref):
    out_ref[...] = jnp.concatenate([u_ref[...], i_ref[...]], axis=0).T


def _make_sc_gather(batch, width, vocab):
    b_per_w = batch // _NW
    n_chunks = b_per_w // _ICHUNK
    mesh = plsc.VectorSubcoreMesh(core_axis_name="c", subcore_axis_name="s")

    @functools.partial(
        pl.kernel,
        mesh=mesh,
        out_type=jax.ShapeDtypeStruct((batch, width), jnp.float32),
        scratch_types=[
            pltpu.VMEM((n_chunks, _ICHUNK), jnp.int32),
            pltpu.VMEM((b_per_w, width), jnp.float32),
            pltpu.SemaphoreType.DMA,
        ],
    )
    def gather_kernel(idx_hbm, tab_hbm, out_hbm, idx_v, rows_v, sem):
        wid = lax.axis_index("s") * _NC + lax.axis_index("c")
        base = wid * b_per_w
        pltpu.sync_copy(idx_hbm.at[wid], idx_v)
        copies = [
            pltpu.make_async_copy(
                tab_hbm.at[idx_v.at[k]],
                rows_v.at[pl.ds(k * _ICHUNK, _ICHUNK)], sem)
            for k in range(n_chunks)
        ]
        for c in copies:
            c.start()
        for c in copies:
            c.wait()
        pltpu.sync_copy(rows_v, out_hbm.at[pl.ds(base, b_per_w)])

    return gather_kernel


def kernel(userIDs, itemIDs, user_table, item_table, W1, b1, W2, b2, W3, b3):
    batch = userIDs.shape[0]
    emb = user_table.shape[1]
    vocab = user_table.shape[0]
    width = 2 * emb
    b_per_w = batch // _NW

    cblk = 8192
    tgrid = (vocab + cblk - 1) // cblk
    cat_rm = pl.pallas_call(
        _transpose_body,
        grid=(tgrid,),
        in_specs=[
            pl.BlockSpec((emb, cblk), lambda i: (0, i)),
            pl.BlockSpec((emb, cblk), lambda i: (0, i)),
        ],
        out_specs=pl.BlockSpec((cblk, width), lambda i: (i, 0)),
        out_shape=jax.ShapeDtypeStruct((vocab, width), jnp.float32),
    )(user_table.T, item_table.T)

    u_idx3 = userIDs.astype(jnp.int32).reshape(_NW, b_per_w // _ICHUNK,
                                               _ICHUNK)
    i_idx3 = itemIDs.astype(jnp.int32).reshape(_NW, b_per_w // _ICHUNK,
                                               _ICHUNK)
    gather = _make_sc_gather(batch, width, vocab)
    u_rows = gather(u_idx3, cat_rm)
    i_rows = gather(i_idx3, cat_rm)

    hid1 = W1.shape[1]
    hid2 = W2.shape[1]

    blk = 2048
    grid = batch // blk
    w1u = jnp.concatenate([W1[:emb], jnp.zeros_like(W1[emb:])], axis=0)
    w1i = jnp.concatenate([jnp.zeros_like(W1[:emb]), W1[emb:]], axis=0)

    def _mlp2_body(u_ref, i_ref, w1u_ref, w1i_ref, b1_ref, w2_ref, b2_ref,
                   w3_ref, b3_ref, out_ref):
        h = jnp.dot(u_ref[...], w1u_ref[...],
                    preferred_element_type=jnp.float32)
        h = h + jnp.dot(i_ref[...], w1i_ref[...],
                        preferred_element_type=jnp.float32)
        h = jnp.maximum(h + b1_ref[...], 0.0)
        h = jnp.maximum(
            jnp.dot(h, w2_ref[...], preferred_element_type=jnp.float32) + b2_ref[...], 0.0)
        out_ref[...] = jnp.dot(
            h, w3_ref[...], preferred_element_type=jnp.float32) + b3_ref[...]

    out = pl.pallas_call(
        _mlp2_body,
        grid=(grid,),
        in_specs=[
            pl.BlockSpec((blk, width), lambda i: (i, 0)),
            pl.BlockSpec((blk, width), lambda i: (i, 0)),
            pl.BlockSpec((width, hid1), lambda i: (0, 0)),
            pl.BlockSpec((width, hid1), lambda i: (0, 0)),
            pl.BlockSpec((1, hid1), lambda i: (0, 0)),
            pl.BlockSpec((hid1, hid2), lambda i: (0, 0)),
            pl.BlockSpec((1, hid2), lambda i: (0, 0)),
            pl.BlockSpec((hid2, 1), lambda i: (0, 0)),
            pl.BlockSpec((1, 1), lambda i: (0, 0)),
        ],
        out_specs=pl.BlockSpec((blk, 1), lambda i: (i, 0)),
        out_shape=jax.ShapeDtypeStruct((batch, 1), jnp.float32),
    )(u_rows, i_rows, w1u, w1i, b1.reshape(1, hid1), W2,
      b2.reshape(1, hid2), W3, b3.reshape(1, 1))
    return out


# speedup vs baseline: 3.1639x; 1.0186x over previous
import functools

import jax
import jax.numpy as jnp
from jax import lax
from jax.experimental import pallas as pl
from jax.experimental.pallas import tpu as pltpu
from jax.experimental.pallas import tpu_sc as plsc

_NC = 2
_NS = 16
_NW = _NC * _NS
_ICHUNK = 128


def _transpose_body(u_ref, i_ref, out_ref):
    out_ref[...] = jnp.concatenate([u_ref[...], i_ref[...]], axis=0).T


def _make_sc_gather(batch, width, vocab):
    b_per_w = batch // _NW
    n_chunks = b_per_w // _ICHUNK
    mesh = plsc.VectorSubcoreMesh(core_axis_name="c", subcore_axis_name="s")

    @functools.partial(
        pl.kernel,
        mesh=mesh,
        out_type=jax.ShapeDtypeStruct((batch, width), jnp.float32),
        scratch_types=[
            pltpu.VMEM((n_chunks, _ICHUNK), jnp.int32),
            pltpu.VMEM((b_per_w, width), jnp.float32),
            pltpu.SemaphoreType.DMA,
        ],
    )
    def gather_kernel(idx_hbm, tab_hbm, out_hbm, idx_v, rows_v, sem):
        wid = lax.axis_index("s") * _NC + lax.axis_index("c")
        base = wid * b_per_w
        pltpu.sync_copy(idx_hbm.at[wid], idx_v)
        copies = [
            pltpu.make_async_copy(
                tab_hbm.at[idx_v.at[k]],
                rows_v.at[pl.ds(k * _ICHUNK, _ICHUNK)], sem)
            for k in range(n_chunks)
        ]
        for c in copies:
            c.start()
        for c in copies:
            c.wait()
        pltpu.sync_copy(rows_v, out_hbm.at[pl.ds(base, b_per_w)])

    return gather_kernel


def kernel(userIDs, itemIDs, user_table, item_table, W1, b1, W2, b2, W3, b3):
    batch = userIDs.shape[0]
    emb = user_table.shape[1]
    vocab = user_table.shape[0]
    width = 2 * emb
    b_per_w = batch // _NW

    cblk = 16384
    tgrid = (vocab + cblk - 1) // cblk
    cat_rm = pl.pallas_call(
        _transpose_body,
        grid=(tgrid,),
        in_specs=[
            pl.BlockSpec((emb, cblk), lambda i: (0, i)),
            pl.BlockSpec((emb, cblk), lambda i: (0, i)),
        ],
        out_specs=pl.BlockSpec((cblk, width), lambda i: (i, 0)),
        out_shape=jax.ShapeDtypeStruct((vocab, width), jnp.float32),
    )(user_table.T, item_table.T)

    u_idx3 = userIDs.astype(jnp.int32).reshape(_NW, b_per_w // _ICHUNK,
                                               _ICHUNK)
    i_idx3 = itemIDs.astype(jnp.int32).reshape(_NW, b_per_w // _ICHUNK,
                                               _ICHUNK)
    gather = _make_sc_gather(batch, width, vocab)
    u_rows = gather(u_idx3, cat_rm)
    i_rows = gather(i_idx3, cat_rm)

    hid1 = W1.shape[1]
    hid2 = W2.shape[1]

    blk = 2048
    grid = batch // blk
    w1u = jnp.concatenate([W1[:emb], jnp.zeros_like(W1[emb:])], axis=0)
    w1i = jnp.concatenate([jnp.zeros_like(W1[:emb]), W1[emb:]], axis=0)

    def _mlp2_body(u_ref, i_ref, w1u_ref, w1i_ref, b1_ref, w2_ref, b2_ref,
                   w3_ref, b3_ref, out_ref):
        h = jnp.dot(u_ref[...], w1u_ref[...],
                    preferred_element_type=jnp.float32)
        h = h + jnp.dot(i_ref[...], w1i_ref[...],
                        preferred_element_type=jnp.float32)
        h = jnp.maximum(h + b1_ref[...], 0.0)
        h = jnp.maximum(
            jnp.dot(h, w2_ref[...], preferred_element_type=jnp.float32) + b2_ref[...], 0.0)
        out_ref[...] = jnp.dot(
            h, w3_ref[...], preferred_element_type=jnp.float32) + b3_ref[...]

    out = pl.pallas_call(
        _mlp2_body,
        grid=(grid,),
        in_specs=[
            pl.BlockSpec((blk, width), lambda i: (i, 0)),
            pl.BlockSpec((blk, width), lambda i: (i, 0)),
            pl.BlockSpec((width, hid1), lambda i: (0, 0)),
            pl.BlockSpec((width, hid1), lambda i: (0, 0)),
            pl.BlockSpec((1, hid1), lambda i: (0, 0)),
            pl.BlockSpec((hid1, hid2), lambda i: (0, 0)),
            pl.BlockSpec((1, hid2), lambda i: (0, 0)),
            pl.BlockSpec((hid2, 1), lambda i: (0, 0)),
            pl.BlockSpec((1, 1), lambda i: (0, 0)),
        ],
        out_specs=pl.BlockSpec((blk, 1), lambda i: (i, 0)),
        out_shape=jax.ShapeDtypeStruct((batch, 1), jnp.float32),
    )(u_rows, i_rows, w1u, w1i, b1.reshape(1, hid1), W2,
      b2.reshape(1, hid2), W3, b3.reshape(1, 1))
    return out


# speedup vs baseline: 3.8102x; 1.2043x over previous
import functools

import jax
import jax.numpy as jnp
from jax import lax
from jax.experimental import pallas as pl
from jax.experimental.pallas import tpu as pltpu
from jax.experimental.pallas import tpu_sc as plsc

_NC = 2
_NS = 16
_NW = _NC * _NS
_ICHUNK = 128


def _round_bf16(cat):
    ci = lax.bitcast_convert_type(cat, jnp.uint32)
    return (ci + jnp.uint32(0x7FFF) + ((ci >> 16) & jnp.uint32(1))) >> 16


def _pack_body(ulo_ref, ilo_ref, uhi_ref, ihi_ref, out_ref):
    cat_lo = jnp.concatenate([ulo_ref[...], ilo_ref[...]], axis=0).T
    cat_hi = jnp.concatenate([uhi_ref[...], ihi_ref[...]], axis=0).T
    packed = (_round_bf16(cat_hi) << 16) | _round_bf16(cat_lo)
    out_ref[...] = lax.bitcast_convert_type(packed, jnp.int32)


def _make_sc_gather(batch, width, npairs):
    b_per_w = batch // _NW
    n_chunks = b_per_w // _ICHUNK
    mesh = plsc.VectorSubcoreMesh(core_axis_name="c", subcore_axis_name="s")

    @functools.partial(
        pl.kernel,
        mesh=mesh,
        out_type=(
            jax.ShapeDtypeStruct((batch, width), jnp.int32),
            jax.ShapeDtypeStruct((batch, width), jnp.int32),
        ),
        scratch_types=[
            pltpu.VMEM((n_chunks, _ICHUNK), jnp.int32),
            pltpu.VMEM((n_chunks, _ICHUNK), jnp.int32),
            pltpu.VMEM((b_per_w, width), jnp.int32),
            pltpu.SemaphoreType.DMA,
        ],
    )
    def gather_kernel(u_idx_hbm, i_idx_hbm, tab_hbm, u_out_hbm, i_out_hbm,
                      uidx_v, iidx_v, rows_v, sem):
        wid = lax.axis_index("s") * _NC + lax.axis_index("c")
        base = wid * b_per_w
        pltpu.sync_copy(u_idx_hbm.at[wid], uidx_v)
        pltpu.sync_copy(i_idx_hbm.at[wid], iidx_v)

        def run_table(idx_v, out_hbm):
            copies = [
                pltpu.make_async_copy(
                    tab_hbm.at[idx_v.at[k]],
                    rows_v.at[pl.ds(k * _ICHUNK, _ICHUNK)], sem)
                for k in range(n_chunks)
            ]
            for c in copies:
                c.start()
            for c in copies:
                c.wait()
            pltpu.sync_copy(rows_v, out_hbm.at[pl.ds(base, b_per_w)])

        run_table(uidx_v, u_out_hbm)
        run_table(iidx_v, i_out_hbm)

    return gather_kernel


def kernel(userIDs, itemIDs, user_table, item_table, W1, b1, W2, b2, W3, b3):
    batch = userIDs.shape[0]
    emb = user_table.shape[1]
    vocab = user_table.shape[0]
    width = 2 * emb
    b_per_w = batch // _NW

    cblk = 16384
    nb = (vocab + 2 * cblk - 1) // (2 * cblk)
    half = nb * cblk
    packed = pl.pallas_call(
        _pack_body,
        grid=(nb,),
        in_specs=[
            pl.BlockSpec((emb, cblk), lambda i: (0, i)),
            pl.BlockSpec((emb, cblk), lambda i: (0, i)),
            pl.BlockSpec((emb, cblk), lambda i: (0, i + nb)),
            pl.BlockSpec((emb, cblk), lambda i: (0, i + nb)),
        ],
        out_specs=pl.BlockSpec((cblk, width), lambda i: (i, 0)),
        out_shape=jax.ShapeDtypeStruct((half, width), jnp.int32),
    )(user_table.T, item_table.T, user_table.T, item_table.T)

    uid = userIDs.astype(jnp.int32)
    iid = itemIDs.astype(jnp.int32)
    up = jnp.where(uid < half, uid, uid - half)
    ip = jnp.where(iid < half, iid, iid - half)
    u_idx3 = up.reshape(_NW, b_per_w // _ICHUNK, _ICHUNK)
    i_idx3 = ip.reshape(_NW, b_per_w // _ICHUNK, _ICHUNK)
    u_pairs, i_pairs = _make_sc_gather(batch, width, half)(
        u_idx3, i_idx3, packed)

    hid1 = W1.shape[1]
    hid2 = W2.shape[1]
    blk = 2048
    grid = batch // blk
    upar = (uid >= half).astype(jnp.int32).reshape(batch, 1)
    ipar = (iid >= half).astype(jnp.int32).reshape(batch, 1)
    w1u = jnp.concatenate([W1[:emb], jnp.zeros_like(W1[emb:])], axis=0)
    w1i = jnp.concatenate([jnp.zeros_like(W1[:emb]), W1[emb:]], axis=0)

    def _mlp_body(u_ref, i_ref, up_ref, ip_ref, w1u_ref, w1i_ref, b1_ref,
                  w2_ref, b2_ref, w3_ref, b3_ref, out_ref):
        def select_unpack(pairs, par):
            pu = lax.bitcast_convert_type(pairs, jnp.uint32)
            v = jnp.where(par != 0, pu >> 16, pu & jnp.uint32(0xFFFF))
            return lax.bitcast_convert_type(v << 16, jnp.float32)

        u = select_unpack(u_ref[...], up_ref[...])
        i = select_unpack(i_ref[...], ip_ref[...])
        h = jnp.dot(u, w1u_ref[...], preferred_element_type=jnp.float32)
        h = h + jnp.dot(i, w1i_ref[...], preferred_element_type=jnp.float32)
        h = jnp.maximum(h + b1_ref[...], 0.0)
        h = jnp.maximum(
            jnp.dot(h, w2_ref[...],
                    preferred_element_type=jnp.float32) + b2_ref[...], 0.0)
        out_ref[...] = jnp.dot(
            h, w3_ref[...], preferred_element_type=jnp.float32) + b3_ref[...]

    out = pl.pallas_call(
        _mlp_body,
        grid=(grid,),
        in_specs=[
            pl.BlockSpec((blk, width), lambda i: (i, 0)),
            pl.BlockSpec((blk, width), lambda i: (i, 0)),
            pl.BlockSpec((blk, 1), lambda i: (i, 0)),
            pl.BlockSpec((blk, 1), lambda i: (i, 0)),
            pl.BlockSpec((width, hid1), lambda i: (0, 0)),
            pl.BlockSpec((width, hid1), lambda i: (0, 0)),
            pl.BlockSpec((1, hid1), lambda i: (0, 0)),
            pl.BlockSpec((hid1, hid2), lambda i: (0, 0)),
            pl.BlockSpec((1, hid2), lambda i: (0, 0)),
            pl.BlockSpec((hid2, 1), lambda i: (0, 0)),
            pl.BlockSpec((1, 1), lambda i: (0, 0)),
        ],
        out_specs=pl.BlockSpec((blk, 1), lambda i: (i, 0)),
        out_shape=jax.ShapeDtypeStruct((batch, 1), jnp.float32),
    )(u_pairs, i_pairs, upar, ipar, w1u, w1i, b1.reshape(1, hid1), W2,
      b2.reshape(1, hid2), W3, b3.reshape(1, 1))
    return out
